# Initial kernel scaffold; baseline (speedup 1.0000x reference)
import jax
import jax.numpy as jnp
from jax.experimental import pallas as pl


def kernel(x, w_g):
    raise NotImplementedError("write your pallas kernel here")


# speedup vs baseline: 1.5965x; 1.5965x over previous
import math

import jax
import jax.numpy as jnp
from jax import lax
from jax.experimental import pallas as pl

NEXP = 8
TOPK = 2
TB = 256


def _gate_kernel(x_ref, w_ref, metaf_ref, metai_ref):
    logitsT = lax.dot_general(
        w_ref[...], x_ref[...], (((1,), (1,)), ((), ())),
        preferred_element_type=jnp.float32)
    iota_e = lax.broadcasted_iota(jnp.int32, (NEXP, TB), 0)
    m1 = jnp.max(logitsT, axis=0, keepdims=True)
    e1 = jnp.min(jnp.where(logitsT == m1, iota_e, NEXP), axis=0, keepdims=True)
    masked = jnp.where(iota_e == e1, -jnp.inf, logitsT)
    m2 = jnp.max(masked, axis=0, keepdims=True)
    e2 = jnp.min(jnp.where(masked == m2, iota_e, NEXP), axis=0, keepdims=True)
    metaf_ref[0] = jnp.concatenate([m1, m2], axis=0)
    metai_ref[0] = jnp.concatenate([e1, e2], axis=0)


def _rank_kernel(metaf_ref, metai_ref, rank_ref, val_ref, used_ref, *, nb, cap):
    ri = lax.broadcasted_iota(jnp.int32, (TB, TB), 0)
    ci = lax.broadcasted_iota(jnp.int32, (TB, TB), 1)
    U = (ri <= ci).astype(jnp.float32)
    iota_e = lax.broadcasted_iota(jnp.int32, (NEXP, TB), 0)

    carry1 = jnp.zeros((NEXP, 1), jnp.float32)
    carry2 = jnp.zeros((NEXP, 1), jnp.float32)
    oh1s, oh2s, cum1s, cum2s = [], [], [], []
    for b in range(nb):
        e1 = metai_ref[b, 0:1, :]
        e2 = metai_ref[b, 1:2, :]
        oh1 = (iota_e == e1)
        oh2 = (iota_e == e2)
        oh1f = oh1.astype(jnp.float32)
        oh2f = oh2.astype(jnp.float32)
        cum1 = lax.dot_general(oh1f, U, (((1,), (0,)), ((), ())),
                               preferred_element_type=jnp.float32) + carry1
        cum2 = lax.dot_general(oh2f, U, (((1,), (0,)), ((), ())),
                               preferred_element_type=jnp.float32) + carry2
        carry1 = cum1[:, TB - 1:TB]
        carry2 = cum2[:, TB - 1:TB]
        oh1s.append((oh1, oh1f))
        oh2s.append((oh2, oh2f))
        cum1s.append(cum1)
        cum2s.append(cum2)

    total1 = carry1
    total2 = carry2
    used = jnp.minimum(total1 + total2, float(cap)).astype(jnp.int32)
    used_ref[...] = used

    for b in range(nb):
        oh1, oh1f = oh1s[b]
        oh2, oh2f = oh2s[b]
        r1 = jnp.sum(oh1f * cum1s[b], axis=0, keepdims=True) - 1.0
        base2 = jnp.sum(oh2f * total1, axis=0, keepdims=True)
        r2 = jnp.sum(oh2f * cum2s[b], axis=0, keepdims=True) - 1.0 + base2
        v1 = metaf_ref[b, 0:1, :]
        v2 = metaf_ref[b, 1:2, :]
        d = jnp.exp(v2 - v1)
        p1 = 1.0 / (1.0 + d)
        p2 = d * p1
        rankT = jnp.where(oh1, r1, jnp.where(oh2, r2, -1.0)).astype(jnp.int32)
        valT = jnp.where(oh1, p1, jnp.where(oh2, p2, 0.0))
        rank_ref[b * TB:(b + 1) * TB, :] = jnp.transpose(rankT, (1, 0))
        val_ref[b * TB:(b + 1) * TB, :] = jnp.transpose(valT, (1, 0))


def _expand_kernel(rank_ref, val_ref, cb_ref, sec_ref, *, cap):
    rank3 = rank_ref[...][:, :, None]
    val3 = val_ref[...][:, :, None]
    iota_cap = lax.broadcasted_iota(jnp.int32, (TB, NEXP, cap), 2)
    eq = rank3 == iota_cap
    cb_ref[...] = jnp.where(eq, val3, 0.0)
    sec_ref[...] = jnp.logical_and(eq, val3 != 0.0)


def kernel(x, w_g):
    Bb, Tt, E = x.shape
    n = Bb * Tt
    nb = n // TB
    cap = math.floor(TOPK * 1.25 * n / NEXP)
    cap += cap % 2
    cap = max(cap, 4)

    x2 = x.reshape(n, E).astype(jnp.float32)

    metaf, metai = pl.pallas_call(
        _gate_kernel,
        grid=(nb,),
        in_specs=[
            pl.BlockSpec((TB, E), lambda i: (i, 0)),
            pl.BlockSpec((NEXP, E), lambda i: (0, 0)),
        ],
        out_specs=[
            pl.BlockSpec((1, TOPK, TB), lambda i: (i, 0, 0)),
            pl.BlockSpec((1, TOPK, TB), lambda i: (i, 0, 0)),
        ],
        out_shape=[
            jax.ShapeDtypeStruct((nb, TOPK, TB), jnp.float32),
            jax.ShapeDtypeStruct((nb, TOPK, TB), jnp.int32),
        ],
    )(x2, w_g)

    import functools
    rank, val, used = pl.pallas_call(
        functools.partial(_rank_kernel, nb=nb, cap=cap),
        out_shape=[
            jax.ShapeDtypeStruct((n, NEXP), jnp.int32),
            jax.ShapeDtypeStruct((n, NEXP), jnp.float32),
            jax.ShapeDtypeStruct((NEXP, 1), jnp.int32),
        ],
    )(metaf, metai)

    cb, sec = pl.pallas_call(
        functools.partial(_expand_kernel, cap=cap),
        grid=(nb,),
        in_specs=[
            pl.BlockSpec((TB, NEXP), lambda i: (i, 0)),
            pl.BlockSpec((TB, NEXP), lambda i: (i, 0)),
        ],
        out_specs=[
            pl.BlockSpec((TB, NEXP, cap), lambda i: (i, 0, 0)),
            pl.BlockSpec((TB, NEXP, cap), lambda i: (i, 0, 0)),
        ],
        out_shape=[
            jax.ShapeDtypeStruct((n, NEXP, cap), jnp.float32),
            jax.ShapeDtypeStruct((n, NEXP, cap), jnp.bool_),
        ],
    )(rank, val)

    return (used.reshape(NEXP), cb, sec)
